# Initial kernel scaffold; baseline (speedup 1.0000x reference)
import jax
import jax.numpy as jnp
from jax.experimental import pallas as pl


def kernel(x, perm_indices):
    raise NotImplementedError("write your pallas kernel here")


# speedup vs baseline: 2.3916x; 2.3916x over previous
import functools

import jax
import jax.numpy as jnp
from jax import lax
from jax.experimental import pallas as pl
from jax.experimental.pallas import tpu as pltpu
from jax.experimental.pallas import tpu_sc as plsc

_B, _S, _D = 4, 4096, 2048
_NC, _NS = 2, 16
_NW = _NC * _NS
_RPW = (_B * _S) // _NW
_K = 16
_NCHUNK = _RPW // _K

_mesh = plsc.VectorSubcoreMesh(core_axis_name="c", subcore_axis_name="s")


@functools.partial(
    pl.kernel,
    mesh=_mesh,
    out_type=jax.ShapeDtypeStruct((_B * _S, _D), jnp.float32),
    scratch_types=[
        pltpu.VMEM((_RPW,), jnp.int32),
        pltpu.VMEM((_K, _D), jnp.float32),
        pltpu.SemaphoreType.DMA,
    ],
)
def _permute_rows(x_hbm, gidx_hbm, out_hbm, idx_v, buf, gsem):
    wid = lax.axis_index("s") * _NC + lax.axis_index("c")
    base = wid * _RPW
    pltpu.sync_copy(gidx_hbm.at[pl.ds(base, _RPW)], idx_v)

    def chunk(c, carry):
        rows = pl.ds(c * _K, _K)
        pltpu.async_copy(x_hbm.at[idx_v.at[rows]], buf, gsem).wait()
        pltpu.sync_copy(buf, out_hbm.at[pl.ds(base + c * _K, _K)])
        return carry

    lax.fori_loop(0, _NCHUNK, chunk, 0)


def kernel(x, perm_indices):
    gidx = (perm_indices.astype(jnp.int32).reshape(1, _S)
            + (jnp.arange(_B, dtype=jnp.int32) * _S).reshape(_B, 1)).reshape(-1)
    out = _permute_rows(x.reshape(_B * _S, _D), gidx)
    return out.reshape(_B, _S, _D)


# speedup vs baseline: 2.8747x; 1.2020x over previous
import functools

import jax
import jax.numpy as jnp
from jax import lax
from jax.experimental import pallas as pl
from jax.experimental.pallas import tpu as pltpu
from jax.experimental.pallas import tpu_sc as plsc

_B, _S, _D = 4, 4096, 2048
_NC, _NS = 2, 16
_NW = _NC * _NS
_RPW = (_B * _S) // _NW
_K = 16
_NCHUNK = _RPW // _K
_NPAIR = _NCHUNK // 2

_mesh = plsc.VectorSubcoreMesh(core_axis_name="c", subcore_axis_name="s")


@functools.partial(
    pl.kernel,
    mesh=_mesh,
    out_type=jax.ShapeDtypeStruct((_B * _S, _D), jnp.float32),
    scratch_types=[
        pltpu.VMEM((_RPW,), jnp.int32),
        pltpu.VMEM((_K, _D), jnp.float32),
        pltpu.VMEM((_K, _D), jnp.float32),
        pltpu.SemaphoreType.DMA,
        pltpu.SemaphoreType.DMA,
        pltpu.SemaphoreType.DMA,
        pltpu.SemaphoreType.DMA,
    ],
)
def _permute_rows(x_hbm, gidx_hbm, out_hbm, idx_v, buf0, buf1, g0, g1, s0, s1):
    wid = lax.axis_index("s") * _NC + lax.axis_index("c")
    base = wid * _RPW
    pltpu.sync_copy(gidx_hbm.at[pl.ds(base, _RPW)], idx_v)

    def gather(c, buf, sem):
        return pltpu.make_async_copy(
            x_hbm.at[idx_v.at[pl.ds(c * _K, _K)]], buf, sem)

    def store(c, buf, sem):
        return pltpu.make_async_copy(
            buf, out_hbm.at[pl.ds(base + c * _K, _K)], sem)

    gather(0, buf0, g0).start()

    def body(i, carry):
        c0 = 2 * i
        c1 = c0 + 1

        @pl.when(i > 0)
        def _():
            store(c1 - 2, buf1, s1).wait()
        gather(c1, buf1, g1).start()

        gather(c0, buf0, g0).wait()
        store(c0, buf0, s0).start()

        @pl.when(i < _NPAIR - 1)
        def _():
            store(c0, buf0, s0).wait()
            gather(c0 + 2, buf0, g0).start()

        gather(c1, buf1, g1).wait()
        store(c1, buf1, s1).start()
        return carry

    lax.fori_loop(0, _NPAIR, body, 0)

    store(_NCHUNK - 2, buf0, s0).wait()
    store(_NCHUNK - 1, buf1, s1).wait()


def kernel(x, perm_indices):
    gidx = (perm_indices.astype(jnp.int32).reshape(1, _S)
            + (jnp.arange(_B, dtype=jnp.int32) * _S).reshape(_B, 1)).reshape(-1)
    out = _permute_rows(x.reshape(_B * _S, _D), gidx)
    return out.reshape(_B, _S, _D)
